# baseline (speedup 1.0000x reference)
import jax
import jax.numpy as jnp
from jax.experimental import pallas as pl


def _gated_body(a1_ref, a2_ref, e_ref, ang_ref, d_ref,
                wf1_ref, wf2_ref, wfe_ref, wfa_ref, bf_ref,
                ws1_ref, ws2_ref, wse_ref, wsa_ref, bs_ref,
                out_ref):
    a1 = a1_ref[...]
    a2 = a2_ref[...]
    e = e_ref[...]
    ang = ang_ref[...]
    pre_f = (jnp.dot(a1, wf1_ref[...], preferred_element_type=jnp.float32)
             + jnp.dot(a2, wf2_ref[...], preferred_element_type=jnp.float32)
             + jnp.dot(e, wfe_ref[...], preferred_element_type=jnp.float32)
             + jnp.dot(ang, wfa_ref[...], preferred_element_type=jnp.float32)
             + bf_ref[...])
    pre_s = (jnp.dot(a1, ws1_ref[...], preferred_element_type=jnp.float32)
             + jnp.dot(a2, ws2_ref[...], preferred_element_type=jnp.float32)
             + jnp.dot(e, wse_ref[...], preferred_element_type=jnp.float32)
             + jnp.dot(ang, wsa_ref[...], preferred_element_type=jnp.float32)
             + bs_ref[...])
    sp = jnp.maximum(pre_s, 0.0) + jnp.log1p(jnp.exp(-jnp.abs(pre_s)))
    gate = jax.nn.sigmoid(pre_f) * sp
    d = d_ref[...]
    expfac = jnp.exp(d * d * (-1.0 / 18.0))
    out_ref[...] = gate * expfac


def _final_body(x0_ref, x1_ref, e_ref, w1a_ref, w1b_ref, w1e_ref, b1_ref,
                w2_ref, b2_ref, out_ref):
    h = (jnp.dot(x0_ref[...], w1a_ref[...], preferred_element_type=jnp.float32)
         + jnp.dot(x1_ref[...], w1b_ref[...], preferred_element_type=jnp.float32)
         + jnp.dot(e_ref[...], w1e_ref[...], preferred_element_type=jnp.float32)
         + b1_ref[...])
    h = h * jax.nn.sigmoid(h)
    out_ref[...] = (jnp.dot(h, w2_ref[...], preferred_element_type=jnp.float32)
                    + b2_ref[...])


def _full_w(shape_nd):
    return pl.BlockSpec(shape_nd, lambda i: tuple(0 for _ in shape_nd))


def kernel(atom_fea, edge_fea, sub_atom_idx, sub_edge_idx, sub_edge_ang,
           sub_index, distance, Wf, bf, Ws, bs, W1, b1, W2, b2):
    n_atom, da = atom_fea.shape
    n_edge, de = edge_fea.shape
    s = sub_edge_idx.shape[0]
    ang = sub_edge_ang.shape[1]
    hid = W1.shape[1]
    dout = W2.shape[1]

    a1 = atom_fea[sub_atom_idx[:, 0]]
    a2 = atom_fea[sub_atom_idx[:, 1]]
    eg = edge_fea[sub_edge_idx]
    dg = distance[sub_edge_idx].reshape(s, 1)

    wf1, wf2, wfe, wfa = Wf[:da], Wf[da:2 * da], Wf[2 * da:2 * da + de], Wf[2 * da + de:]
    ws1, ws2, wse, wsa = Ws[:da], Ws[da:2 * da], Ws[2 * da:2 * da + de], Ws[2 * da + de:]

    bs_blk = 2000
    grid = (s // bs_blk,)
    row = lambda i: (i, 0)
    gated = pl.pallas_call(
        _gated_body,
        grid=grid,
        in_specs=[
            pl.BlockSpec((bs_blk, da), row),
            pl.BlockSpec((bs_blk, da), row),
            pl.BlockSpec((bs_blk, de), row),
            pl.BlockSpec((bs_blk, ang), row),
            pl.BlockSpec((bs_blk, 1), row),
            _full_w((da, da)), _full_w((da, da)), _full_w((de, da)),
            _full_w((ang, da)), _full_w((da,)),
            _full_w((da, da)), _full_w((da, da)), _full_w((de, da)),
            _full_w((ang, da)), _full_w((da,)),
        ],
        out_specs=pl.BlockSpec((bs_blk, da), row),
        out_shape=jax.ShapeDtypeStruct((s, da), jnp.float32),
    )(a1, a2, eg, sub_edge_ang, dg,
      wf1, wf2, wfe, wfa, bf, ws1, ws2, wse, wsa, bs)

    seg = jax.ops.segment_sum(gated, sub_index, num_segments=2 * n_edge)
    seg = seg.reshape(n_edge, 2, da)
    x0 = seg[:, 0, :]
    x1 = seg[:, 1, :]

    w1a, w1b, w1e = W1[:da], W1[da:2 * da], W1[2 * da:]
    eb = 2000
    grid2 = (n_edge // eb,)
    out = pl.pallas_call(
        _final_body,
        grid=grid2,
        in_specs=[
            pl.BlockSpec((eb, da), row),
            pl.BlockSpec((eb, da), row),
            pl.BlockSpec((eb, de), row),
            _full_w((da, hid)), _full_w((da, hid)), _full_w((de, hid)),
            _full_w((hid,)),
            _full_w((hid, dout)), _full_w((dout,)),
        ],
        out_specs=pl.BlockSpec((eb, dout), row),
        out_shape=jax.ShapeDtypeStruct((n_edge, dout), jnp.float32),
    )(x0, x1, edge_fea, w1a, w1b, w1e, b1, W2, b2)

    return out


# speedup vs baseline: 2.4412x; 2.4412x over previous
import functools

import jax
import jax.numpy as jnp
from jax import lax
from jax.experimental import pallas as pl
from jax.experimental.pallas import tpu as pltpu
from jax.experimental.pallas import tpu_sc as plsc

_NC = 2
_NS = 16
_NW = _NC * _NS
_C = 128


def _sc_gather_body(atom_hbm, edge_hbm, dist_hbm, i0_hbm, i1_hbm, ie_hbm,
                    a1_out, a2_out, eg_out, dg_out,
                    i0_v, i1_v, ie_v, r1_v, r2_v, re_v, rd_v, sem):
    wid = lax.axis_index("s") * _NC + lax.axis_index("c")
    n_chunks = i0_hbm.shape[0]
    n_rounds = (n_chunks + _NW - 1) // _NW

    def body(j, _):
        cid = j * _NW + wid

        @pl.when(cid < n_chunks)
        def _():
            pltpu.sync_copy(i0_hbm.at[cid], i0_v)
            pltpu.sync_copy(i1_hbm.at[cid], i1_v)
            pltpu.sync_copy(ie_hbm.at[cid], ie_v)
            c1 = pltpu.async_copy(atom_hbm.at[i0_v], r1_v, sem)
            c2 = pltpu.async_copy(atom_hbm.at[i1_v], r2_v, sem)
            c3 = pltpu.async_copy(edge_hbm.at[ie_v], re_v, sem)
            c4 = pltpu.async_copy(dist_hbm.at[ie_v], rd_v, sem)
            c1.wait()
            c2.wait()
            c3.wait()
            c4.wait()
            base = cid * _C
            pltpu.sync_copy(r1_v, a1_out.at[pl.ds(base, _C)])
            pltpu.sync_copy(r2_v, a2_out.at[pl.ds(base, _C)])
            pltpu.sync_copy(re_v, eg_out.at[pl.ds(base, _C)])
            pltpu.sync_copy(rd_v, dg_out.at[cid])
        return 0

    lax.fori_loop(0, n_rounds, body, 0)


def _gated_body(a1_ref, a2_ref, e_ref, ang_ref, d_ref,
                wf1_ref, wf2_ref, wfe_ref, wfa_ref, bf_ref,
                ws1_ref, ws2_ref, wse_ref, wsa_ref, bs_ref,
                out_ref):
    a1 = a1_ref[...]
    a2 = a2_ref[...]
    e = e_ref[...]
    ang = ang_ref[...]
    pre_f = (jnp.dot(a1, wf1_ref[...], preferred_element_type=jnp.float32)
             + jnp.dot(a2, wf2_ref[...], preferred_element_type=jnp.float32)
             + jnp.dot(e, wfe_ref[...], preferred_element_type=jnp.float32)
             + jnp.dot(ang, wfa_ref[...], preferred_element_type=jnp.float32)
             + bf_ref[...])
    pre_s = (jnp.dot(a1, ws1_ref[...], preferred_element_type=jnp.float32)
             + jnp.dot(a2, ws2_ref[...], preferred_element_type=jnp.float32)
             + jnp.dot(e, wse_ref[...], preferred_element_type=jnp.float32)
             + jnp.dot(ang, wsa_ref[...], preferred_element_type=jnp.float32)
             + bs_ref[...])
    sp = jnp.maximum(pre_s, 0.0) + jnp.log1p(jnp.exp(-jnp.abs(pre_s)))
    gate = jax.nn.sigmoid(pre_f) * sp
    d = d_ref[...]
    expfac = jnp.exp(d * d * (-1.0 / 18.0))
    out_ref[...] = gate * expfac


def _final_body(x0_ref, x1_ref, e_ref, w1a_ref, w1b_ref, w1e_ref, b1_ref,
                w2_ref, b2_ref, out_ref):
    h = (jnp.dot(x0_ref[...], w1a_ref[...], preferred_element_type=jnp.float32)
         + jnp.dot(x1_ref[...], w1b_ref[...], preferred_element_type=jnp.float32)
         + jnp.dot(e_ref[...], w1e_ref[...], preferred_element_type=jnp.float32)
         + b1_ref[...])
    h = h * jax.nn.sigmoid(h)
    out_ref[...] = (jnp.dot(h, w2_ref[...], preferred_element_type=jnp.float32)
                    + b2_ref[...])


def _full_w(shape_nd):
    return pl.BlockSpec(shape_nd, lambda i: tuple(0 for _ in shape_nd))


def kernel(atom_fea, edge_fea, sub_atom_idx, sub_edge_idx, sub_edge_ang,
           sub_index, distance, Wf, bf, Ws, bs, W1, b1, W2, b2):
    n_atom, da = atom_fea.shape
    n_edge, de = edge_fea.shape
    s = sub_edge_idx.shape[0]
    ang = sub_edge_ang.shape[1]
    hid = W1.shape[1]
    dout = W2.shape[1]
    n_chunks = s // _C

    i0 = sub_atom_idx[:, 0].reshape(n_chunks, _C)
    i1 = sub_atom_idx[:, 1].reshape(n_chunks, _C)
    ie = sub_edge_idx.reshape(n_chunks, _C)

    mesh = plsc.VectorSubcoreMesh(core_axis_name="c", subcore_axis_name="s",
                                  num_cores=_NC, num_subcores=_NS)
    gather_k = pl.kernel(
        _sc_gather_body,
        out_type=(
            jax.ShapeDtypeStruct((s, da), jnp.float32),
            jax.ShapeDtypeStruct((s, da), jnp.float32),
            jax.ShapeDtypeStruct((s, de), jnp.float32),
            jax.ShapeDtypeStruct((n_chunks, _C), jnp.float32),
        ),
        mesh=mesh,
        compiler_params=pltpu.CompilerParams(use_tc_tiling_on_sc=False),
        scratch_types=[
            pltpu.VMEM((_C,), jnp.int32),
            pltpu.VMEM((_C,), jnp.int32),
            pltpu.VMEM((_C,), jnp.int32),
            pltpu.VMEM((_C, da), jnp.float32),
            pltpu.VMEM((_C, da), jnp.float32),
            pltpu.VMEM((_C, de), jnp.float32),
            pltpu.VMEM((_C,), jnp.float32),
            pltpu.SemaphoreType.DMA,
        ],
    )
    a1, a2, eg, dg = gather_k(atom_fea, edge_fea, distance, i0, i1, ie)
    dg = dg.reshape(s, 1)

    wf1, wf2, wfe, wfa = Wf[:da], Wf[da:2 * da], Wf[2 * da:2 * da + de], Wf[2 * da + de:]
    ws1, ws2, wse, wsa = Ws[:da], Ws[da:2 * da], Ws[2 * da:2 * da + de], Ws[2 * da + de:]

    bs_blk = 2000
    grid = (s // bs_blk,)
    row = lambda i: (i, 0)
    gated = pl.pallas_call(
        _gated_body,
        grid=grid,
        in_specs=[
            pl.BlockSpec((bs_blk, da), row),
            pl.BlockSpec((bs_blk, da), row),
            pl.BlockSpec((bs_blk, de), row),
            pl.BlockSpec((bs_blk, ang), row),
            pl.BlockSpec((bs_blk, 1), row),
            _full_w((da, da)), _full_w((da, da)), _full_w((de, da)),
            _full_w((ang, da)), _full_w((da,)),
            _full_w((da, da)), _full_w((da, da)), _full_w((de, da)),
            _full_w((ang, da)), _full_w((da,)),
        ],
        out_specs=pl.BlockSpec((bs_blk, da), row),
        out_shape=jax.ShapeDtypeStruct((s, da), jnp.float32),
    )(a1, a2, eg, sub_edge_ang, dg,
      wf1, wf2, wfe, wfa, bf, ws1, ws2, wse, wsa, bs)

    seg = jax.ops.segment_sum(gated, sub_index, num_segments=2 * n_edge)
    seg = seg.reshape(n_edge, 2, da)
    x0 = seg[:, 0, :]
    x1 = seg[:, 1, :]

    w1a, w1b, w1e = W1[:da], W1[da:2 * da], W1[2 * da:]
    eb = 2000
    grid2 = (n_edge // eb,)
    out = pl.pallas_call(
        _final_body,
        grid=grid2,
        in_specs=[
            pl.BlockSpec((eb, da), row),
            pl.BlockSpec((eb, da), row),
            pl.BlockSpec((eb, de), row),
            _full_w((da, hid)), _full_w((da, hid)), _full_w((de, hid)),
            _full_w((hid,)),
            _full_w((hid, dout)), _full_w((dout,)),
        ],
        out_specs=pl.BlockSpec((eb, dout), row),
        out_shape=jax.ShapeDtypeStruct((n_edge, dout), jnp.float32),
    )(x0, x1, edge_fea, w1a, w1b, w1e, b1, W2, b2)

    return out
